# Initial kernel scaffold; baseline (speedup 1.0000x reference)
import jax
import jax.numpy as jnp
from jax.experimental import pallas as pl


def kernel(x, x_v, W1_w, W1_b, V_w, V_b):
    raise NotImplementedError("write your pallas kernel here")


# speedup vs baseline: 4.4845x; 4.4845x over previous
import functools

import jax
import jax.numpy as jnp
from jax.experimental import pallas as pl
from jax.experimental.pallas import tpu as pltpu

_POOL_FRAC = 0.125
_KNN = 27
_KPAD = 32


def _scores_body(x_ref, w1_ref, b1_ref, v_ref, bv_ref, s_col_ref, s_row_ref):
    x = x_ref[0]
    h = jnp.tanh(jnp.dot(x, w1_ref[...]) + b1_ref[...])
    s = jax.nn.sigmoid(jnp.dot(h, v_ref[...]) + bv_ref[...])
    s_col_ref[0] = s
    s_row_ref[0] = s.reshape(1, -1)


def _select_body(s_row_ref, xv_ref, xvn_ref, *, n, ns):
    srow = s_row_ref[0]
    iota_row = jax.lax.broadcasted_iota(jnp.int32, (1, n), 1)
    acc = jnp.zeros((ns, 3), jnp.float32)
    r_col = jax.lax.broadcasted_iota(jnp.int32, (ns, 1), 0)
    for c in range(n // ns):
        sc_col = srow[:, c * ns:(c + 1) * ns].reshape(ns, 1)
        ic_col = jax.lax.broadcasted_iota(jnp.int32, (ns, 1), 0) + c * ns
        beats = (srow > sc_col) | ((srow == sc_col) & (iota_row < ic_col))
        rank = jnp.sum(beats.astype(jnp.int32), axis=1, keepdims=True)
        eq_t = (r_col == rank.reshape(1, ns)).astype(jnp.float32)
        acc = acc + jax.lax.dot_general(
            eq_t, xv_ref[0, c * ns:(c + 1) * ns, :], (((1,), (0,)), ((), ())),
            precision=jax.lax.Precision.HIGHEST)
    xvn_ref[0] = acc


def _norm_long(p):
    return (p[:, 0:1] * p[:, 0:1] + p[:, 1:2] * p[:, 1:2]) + p[:, 2:3] * p[:, 2:3]


def _norm_short(p):
    return (p[:, 0:1] * p[:, 0:1] + p[:, 2:3] * p[:, 2:3]) + p[:, 1:2] * p[:, 1:2]


def _pool_body(xvn_ref, xv_ref, s_row_ref, x_ref, pidx_ref, out_ref,
               d_ref, a_ref, *, n, ns, k):
    p1 = xvn_ref[0]
    p2 = xv_ref[0]
    n1 = _norm_short(p1)
    n2 = _norm_long(p2).reshape(1, n)
    e = jax.lax.dot_general(p1, p2, (((1,), (1,)), ((), ())))
    d_ref[...] = n1 + n2 - 2.0 * e
    a_ref[...] = jnp.zeros((ns, n), jnp.float32)
    iota_n = jax.lax.broadcasted_iota(jnp.int32, (1, n), 1)

    def body(kk, carry):
        d = d_ref[...]
        m = jnp.min(d, axis=1, keepdims=True)
        cand = jnp.where(d == m, iota_n, n)
        idx = jnp.min(cand, axis=1, keepdims=True)
        mask = iota_n == idx
        d_ref[...] = jnp.where(mask, jnp.inf, d)
        a_ref[...] = jnp.where(mask, s_row_ref[0], a_ref[...])
        pidx_ref[0, pl.ds(kk, 1), :] = idx.reshape(1, ns)
        return carry

    jax.lax.fori_loop(0, k, body, 0)
    a = a_ref[...]
    denom = jnp.sum(a, axis=1, keepdims=True)
    acc = jax.lax.dot_general(a, x_ref[0], (((1,), (0,)), ((), ())),
                              precision=jax.lax.Precision.HIGHEST)
    out_ref[0] = acc / denom


def _unpool_body(xv_ref, xvn_ref, uidx_ref, d_ref, *, n, ns, k):
    p1 = xv_ref[0]
    p2 = xvn_ref[0]
    n1 = _norm_long(p1)
    n2 = _norm_short(p2).reshape(1, ns)
    e = jax.lax.dot_general(p1, p2, (((1,), (1,)), ((), ())))
    d_ref[...] = n1 + n2 - 2.0 * e
    iota_s = jax.lax.broadcasted_iota(jnp.int32, (1, ns), 1)

    def body(kk, carry):
        d = d_ref[...]
        m = jnp.min(d, axis=1, keepdims=True)
        cand = jnp.where(d == m, iota_s, ns)
        idx = jnp.min(cand, axis=1, keepdims=True)
        mask = iota_s == idx
        d_ref[...] = jnp.where(mask, jnp.inf, d)
        uidx_ref[0, pl.ds(kk, 1), :] = idx.reshape(1, n)
        return carry

    jax.lax.fori_loop(0, k, body, 0)


def kernel(x, x_v, W1_w, W1_b, V_w, V_b):
    B, N, F = x.shape
    H = W1_w.shape[1]
    NS = int(N * _POOL_FRAC)

    b1 = W1_b.reshape(1, H)
    bv = V_b.reshape(1, 1)

    s_col, s_row = pl.pallas_call(
        _scores_body,
        grid=(B,),
        in_specs=[
            pl.BlockSpec((1, N, F), lambda b: (b, 0, 0)),
            pl.BlockSpec((F, H), lambda b: (0, 0)),
            pl.BlockSpec((1, H), lambda b: (0, 0)),
            pl.BlockSpec((H, 1), lambda b: (0, 0)),
            pl.BlockSpec((1, 1), lambda b: (0, 0)),
        ],
        out_specs=[
            pl.BlockSpec((1, N, 1), lambda b: (b, 0, 0)),
            pl.BlockSpec((1, 1, N), lambda b: (b, 0, 0)),
        ],
        out_shape=[
            jax.ShapeDtypeStruct((B, N, 1), jnp.float32),
            jax.ShapeDtypeStruct((B, 1, N), jnp.float32),
        ],
    )(x, W1_w, b1, V_w, bv)

    x_v_next = pl.pallas_call(
        functools.partial(_select_body, n=N, ns=NS),
        grid=(B,),
        in_specs=[
            pl.BlockSpec((1, 1, N), lambda b: (b, 0, 0)),
            pl.BlockSpec((1, N, 3), lambda b: (b, 0, 0)),
        ],
        out_specs=pl.BlockSpec((1, NS, 3), lambda b: (b, 0, 0)),
        out_shape=jax.ShapeDtypeStruct((B, NS, 3), jnp.float32),
    )(s_row, x_v)

    pidx_t, out = pl.pallas_call(
        functools.partial(_pool_body, n=N, ns=NS, k=_KNN),
        grid=(B,),
        in_specs=[
            pl.BlockSpec((1, NS, 3), lambda b: (b, 0, 0)),
            pl.BlockSpec((1, N, 3), lambda b: (b, 0, 0)),
            pl.BlockSpec((1, 1, N), lambda b: (b, 0, 0)),
            pl.BlockSpec((1, N, F), lambda b: (b, 0, 0)),
        ],
        out_specs=[
            pl.BlockSpec((1, _KPAD, NS), lambda b: (b, 0, 0)),
            pl.BlockSpec((1, NS, F), lambda b: (b, 0, 0)),
        ],
        out_shape=[
            jax.ShapeDtypeStruct((B, _KPAD, NS), jnp.int32),
            jax.ShapeDtypeStruct((B, NS, F), jnp.float32),
        ],
        scratch_shapes=[
            pltpu.VMEM((NS, N), jnp.float32),
            pltpu.VMEM((NS, N), jnp.float32),
        ],
    )(x_v_next, x_v, s_row, x)

    uidx_t = pl.pallas_call(
        functools.partial(_unpool_body, n=N, ns=NS, k=_KNN),
        grid=(B,),
        in_specs=[
            pl.BlockSpec((1, N, 3), lambda b: (b, 0, 0)),
            pl.BlockSpec((1, NS, 3), lambda b: (b, 0, 0)),
        ],
        out_specs=pl.BlockSpec((1, _KPAD, N), lambda b: (b, 0, 0)),
        out_shape=jax.ShapeDtypeStruct((B, _KPAD, N), jnp.int32),
        scratch_shapes=[pltpu.VMEM((N, NS), jnp.float32)],
    )(x_v, x_v_next)

    pooling_idx = jnp.swapaxes(pidx_t[:, :_KNN, :], 1, 2)
    unpooling_idx = jnp.swapaxes(uidx_t[:, :_KNN, :], 1, 2)
    return out, x_v_next, s_col, pooling_idx, unpooling_idx


# speedup vs baseline: 6.7378x; 1.5024x over previous
import functools

import jax
import jax.numpy as jnp
from jax.experimental import pallas as pl
from jax.experimental.pallas import tpu as pltpu

_POOL_FRAC = 0.125
_KNN = 27
_KPAD = 32


def _scores_body(x_ref, w1_ref, b1_ref, v_ref, bv_ref, s_col_ref, s_row_ref):
    x = x_ref[0]
    h = jnp.tanh(jnp.dot(x, w1_ref[...]) + b1_ref[...])
    s = jax.nn.sigmoid(jnp.dot(h, v_ref[...]) + bv_ref[...])
    s_col_ref[0] = s
    s_row_ref[0] = s.reshape(1, -1)


def _select_body(s_row_ref, xv_ref, xvn_ref, *, n, ns):
    srow = s_row_ref[0]
    iota_row = jax.lax.broadcasted_iota(jnp.int32, (1, n), 1)
    acc = jnp.zeros((ns, 3), jnp.float32)
    r_col = jax.lax.broadcasted_iota(jnp.int32, (ns, 1), 0)
    for c in range(n // ns):
        sc_col = srow[:, c * ns:(c + 1) * ns].reshape(ns, 1)
        ic_col = jax.lax.broadcasted_iota(jnp.int32, (ns, 1), 0) + c * ns
        beats = (srow > sc_col) | ((srow == sc_col) & (iota_row < ic_col))
        rank = jnp.sum(beats.astype(jnp.int32), axis=1, keepdims=True)
        eq_t = (r_col == rank.reshape(1, ns)).astype(jnp.float32)
        acc = acc + jax.lax.dot_general(
            eq_t, xv_ref[0, c * ns:(c + 1) * ns, :], (((1,), (0,)), ((), ())),
            precision=jax.lax.Precision.HIGHEST)
    xvn_ref[0] = acc


def _norm_long(p):
    return (p[:, 0:1] * p[:, 0:1] + p[:, 1:2] * p[:, 1:2]) + p[:, 2:3] * p[:, 2:3]


def _norm_short(p):
    return (p[:, 0:1] * p[:, 0:1] + p[:, 2:3] * p[:, 2:3]) + p[:, 1:2] * p[:, 1:2]


def _pool_body(xvn_ref, xv_ref, s_row_ref, x_ref, pidx_ref, out_ref,
               d_ref, *, n, ns, k):
    p1 = xvn_ref[0]
    p2 = xv_ref[0]
    n1 = _norm_short(p1)
    n2 = _norm_long(p2).reshape(1, n)
    e = jax.lax.dot_general(p1, p2, (((1,), (1,)), ((), ())))
    d_ref[...] = n1 + n2 - 2.0 * e
    iota_n = jax.lax.broadcasted_iota(jnp.int32, (1, n), 1)

    def body(kk, carry):
        d = d_ref[...]
        m = jnp.min(d, axis=1, keepdims=True)
        cand = jnp.where(d == m, iota_n, n)
        idx = jnp.min(cand, axis=1, keepdims=True)
        mask = iota_n == idx
        d_ref[...] = jnp.where(mask, jnp.inf, d)
        pidx_ref[0, pl.ds(kk, 1), :] = idx.reshape(1, ns)
        return carry

    jax.lax.fori_loop(0, k, body, 0)
    a = jnp.where(jnp.isinf(d_ref[...]), s_row_ref[0], 0.0)
    denom = jnp.sum(a, axis=1, keepdims=True)
    acc = jax.lax.dot_general(a, x_ref[0], (((1,), (0,)), ((), ())),
                              precision=jax.lax.Precision.HIGHEST)
    out_ref[0] = acc / denom


def _unpool_body(xv_ref, xvn_ref, uidx_ref, d_ref, *, n, ns, k):
    p1 = xvn_ref[0]
    p2 = xv_ref[0]
    n1 = _norm_short(p1)
    n2 = _norm_long(p2).reshape(1, n)
    e = jax.lax.dot_general(p1, p2, (((1,), (1,)), ((), ())))
    d_ref[...] = n1 + n2 - 2.0 * e
    iota_s = jax.lax.broadcasted_iota(jnp.int32, (ns, 1), 0)

    def body(kk, carry):
        d = d_ref[...]
        m = jnp.min(d, axis=0, keepdims=True)
        cand = jnp.where(d == m, iota_s, ns)
        idx = jnp.min(cand, axis=0, keepdims=True)
        mask = iota_s == idx
        d_ref[...] = jnp.where(mask, jnp.inf, d)
        uidx_ref[0, pl.ds(kk, 1), :] = idx
        return carry

    jax.lax.fori_loop(0, k, body, 0)


def kernel(x, x_v, W1_w, W1_b, V_w, V_b):
    B, N, F = x.shape
    H = W1_w.shape[1]
    NS = int(N * _POOL_FRAC)

    b1 = W1_b.reshape(1, H)
    bv = V_b.reshape(1, 1)

    s_col, s_row = pl.pallas_call(
        _scores_body,
        grid=(B,),
        in_specs=[
            pl.BlockSpec((1, N, F), lambda b: (b, 0, 0)),
            pl.BlockSpec((F, H), lambda b: (0, 0)),
            pl.BlockSpec((1, H), lambda b: (0, 0)),
            pl.BlockSpec((H, 1), lambda b: (0, 0)),
            pl.BlockSpec((1, 1), lambda b: (0, 0)),
        ],
        out_specs=[
            pl.BlockSpec((1, N, 1), lambda b: (b, 0, 0)),
            pl.BlockSpec((1, 1, N), lambda b: (b, 0, 0)),
        ],
        out_shape=[
            jax.ShapeDtypeStruct((B, N, 1), jnp.float32),
            jax.ShapeDtypeStruct((B, 1, N), jnp.float32),
        ],
    )(x, W1_w, b1, V_w, bv)

    x_v_next = pl.pallas_call(
        functools.partial(_select_body, n=N, ns=NS),
        grid=(B,),
        in_specs=[
            pl.BlockSpec((1, 1, N), lambda b: (b, 0, 0)),
            pl.BlockSpec((1, N, 3), lambda b: (b, 0, 0)),
        ],
        out_specs=pl.BlockSpec((1, NS, 3), lambda b: (b, 0, 0)),
        out_shape=jax.ShapeDtypeStruct((B, NS, 3), jnp.float32),
    )(s_row, x_v)

    pidx_t, out = pl.pallas_call(
        functools.partial(_pool_body, n=N, ns=NS, k=_KNN),
        grid=(B,),
        in_specs=[
            pl.BlockSpec((1, NS, 3), lambda b: (b, 0, 0)),
            pl.BlockSpec((1, N, 3), lambda b: (b, 0, 0)),
            pl.BlockSpec((1, 1, N), lambda b: (b, 0, 0)),
            pl.BlockSpec((1, N, F), lambda b: (b, 0, 0)),
        ],
        out_specs=[
            pl.BlockSpec((1, _KPAD, NS), lambda b: (b, 0, 0)),
            pl.BlockSpec((1, NS, F), lambda b: (b, 0, 0)),
        ],
        out_shape=[
            jax.ShapeDtypeStruct((B, _KPAD, NS), jnp.int32),
            jax.ShapeDtypeStruct((B, NS, F), jnp.float32),
        ],
        scratch_shapes=[
            pltpu.VMEM((NS, N), jnp.float32),
        ],
    )(x_v_next, x_v, s_row, x)

    uidx_t = pl.pallas_call(
        functools.partial(_unpool_body, n=N, ns=NS, k=_KNN),
        grid=(B,),
        in_specs=[
            pl.BlockSpec((1, N, 3), lambda b: (b, 0, 0)),
            pl.BlockSpec((1, NS, 3), lambda b: (b, 0, 0)),
        ],
        out_specs=pl.BlockSpec((1, _KPAD, N), lambda b: (b, 0, 0)),
        out_shape=jax.ShapeDtypeStruct((B, _KPAD, N), jnp.int32),
        scratch_shapes=[pltpu.VMEM((NS, N), jnp.float32)],
    )(x_v, x_v_next)

    pooling_idx = jnp.swapaxes(pidx_t[:, :_KNN, :], 1, 2)
    unpooling_idx = jnp.swapaxes(uidx_t[:, :_KNN, :], 1, 2)
    return out, x_v_next, s_col, pooling_idx, unpooling_idx
